# baseline (speedup 1.0000x reference)
import functools

import jax
import jax.numpy as jnp
from jax.experimental import pallas as pl
from jax.experimental.pallas import tpu as pltpu

NE = 64
TOPK = 8
HID = 4096
LBW = 0.01
CH = 128


def _gate_kernel(ntok, x_ref, w_ref, ids_ref, gw_ref, aux_ref, cnt_acc, p_acc):
    step = pl.program_id(0)
    nsteps = pl.num_programs(0)

    @pl.when(step == 0)
    def _init():
        cnt_acc[...] = jnp.zeros_like(cnt_acc)
        p_acc[...] = jnp.zeros_like(p_acc)

    x = x_ref[...]
    w = w_ref[...]
    logits = jax.lax.dot_general(
        w, x, (((1,), (1,)), ((), ())),
        preferred_element_type=jnp.float32)

    bt = x.shape[0]
    iota = jax.lax.broadcasted_iota(jnp.int32, (NE, CH), 0)
    for c in range(bt // CH):
        lg = jax.lax.slice(logits, (0, c * CH), (NE, (c + 1) * CH))
        m = jnp.max(lg, axis=0, keepdims=True)
        e = jnp.exp(lg - m)
        s = jnp.sum(e, axis=0, keepdims=True)
        pr = e / s

        p_acc[:, c * CH:(c + 1) * CH] += pr

        running = pr
        rows_id, rows_w = [], []
        mx = None
        for _ in range(TOPK):
            mx = jnp.max(running, axis=0, keepdims=True)
            cand = jnp.where(running == mx, iota, NE)
            sel = jnp.min(cand, axis=0, keepdims=True)
            rows_id.append(sel)
            rows_w.append(mx)
            running = jnp.where(iota == sel, -1.0, running)

        cnt_acc[:, c * CH:(c + 1) * CH] += (pr >= mx).astype(jnp.float32)

        ids = jnp.concatenate(rows_id, axis=0)
        ws = jnp.concatenate(rows_w, axis=0)
        wsum = jnp.sum(ws, axis=0, keepdims=True) + 1e-9
        ids_ref[:, c * CH:(c + 1) * CH] = ids
        gw_ref[:, c * CH:(c + 1) * CH] = ws / wsum

    @pl.when(step == nsteps - 1)
    def _fini():
        counts = jnp.sum(cnt_acc[...], axis=1, keepdims=True)
        psum = jnp.sum(p_acc[...], axis=1, keepdims=True)
        f = counts / (ntok * TOPK)
        p_mean = psum / ntok
        aux_ref[...] = LBW * NE * jnp.sum(f * p_mean, axis=0, keepdims=True)


def _router(x, W, block_tokens, interpret=False):
    T = x.shape[0]
    nb = T // block_tokens
    return pl.pallas_call(
        functools.partial(_gate_kernel, T),
        grid=(nb,),
        in_specs=[
            pl.BlockSpec((block_tokens, HID), lambda i: (i, 0)),
            pl.BlockSpec((NE, HID), lambda i: (0, 0)),
        ],
        out_specs=[
            pl.BlockSpec((TOPK, block_tokens), lambda i: (0, i)),
            pl.BlockSpec((TOPK, block_tokens), lambda i: (0, i)),
            pl.BlockSpec((1, 1), lambda i: (0, 0)),
        ],
        out_shape=[
            jax.ShapeDtypeStruct((TOPK, T), jnp.int32),
            jax.ShapeDtypeStruct((TOPK, T), jnp.float32),
            jax.ShapeDtypeStruct((1, 1), jnp.float32),
        ],
        scratch_shapes=[
            pltpu.VMEM((NE, block_tokens), jnp.float32),
            pltpu.VMEM((NE, block_tokens), jnp.float32),
        ],
        compiler_params=pltpu.CompilerParams(
            dimension_semantics=("arbitrary",),
        ),
        interpret=interpret,
    )(x, W)


def kernel(hidden_states, W):
    x = hidden_states.reshape(-1, HID)
    T = x.shape[0]
    ids_t, gw_t, aux = _router(x, W, block_tokens=1024)
    expert_ids = ids_t.T.reshape(-1)
    gate_weights = gw_t.T.reshape(-1)
    token_indices = jax.lax.broadcasted_iota(jnp.int32, (T, TOPK), 0).reshape(-1)
    return expert_ids, gate_weights, token_indices, aux[0, 0]
